# Initial kernel scaffold; baseline (speedup 1.0000x reference)
import jax
import jax.numpy as jnp
from jax.experimental import pallas as pl


def kernel(x_enc0, x_enc1, x_enc2, x_enc3, x_enc4, x_enc5, edge_index1, edge_index2, edge_index3, edge_index4, edge_index5, W1_1, W1_2, S1, W2_1, W2_2, S2, W3_1, W3_2, S3, W4_1, W4_2, S4, W5_1, W5_2, S5, Wc):
    raise NotImplementedError("write your pallas kernel here")


# speedup vs baseline: 2.1545x; 2.1545x over previous
import jax
import jax.numpy as jnp
from jax.experimental import pallas as pl

L_ORI = 6
N_LVL = [1536, 3072, 6144, 12288, 24576, 49152]


def _cheb_restruct(x, src, dst, dinv, n, W):
    C = W.shape[2]
    Wcat = jnp.concatenate([W[0], W[1], W[2]], axis=1)
    Y = x @ Wcat
    y0, y1, y2 = Y[:, :C], Y[:, C:2*C], Y[:, 2*C:]
    def A(v):
        return jax.ops.segment_sum(v[src], dst, num_segments=n)
    z = A(dinv[:, None] * y2)
    u_in = y1 - 2.0 * dinv[:, None] * z
    u = -dinv[:, None] * A(dinv[:, None] * u_in)
    return y0 - y2 + u


def _resblock(x, ei, n, W1, W2, S):
    src, dst = ei[0], ei[1]
    deg = jax.ops.segment_sum(jnp.ones(src.shape[0], jnp.float32), dst, num_segments=n)
    dinv = jnp.where(deg > 0, 1.0 / jnp.sqrt(jnp.maximum(deg, 1e-12)), 0.0)
    h = jax.nn.relu(_cheb_restruct(x, src, dst, dinv, n, W1))
    g = _cheb_restruct(h, src, dst, dinv, n, W2)
    return jax.nn.relu(g + x @ S)


def kernel(x_enc0, x_enc1, x_enc2, x_enc3, x_enc4, x_enc5,
           edge_index1, edge_index2, edge_index3, edge_index4, edge_index5,
           W1_1, W1_2, S1, W2_1, W2_2, S2, W3_1, W3_2, S3,
           W4_1, W4_2, S4, W5_1, W5_2, S5, Wc):
    t = lambda a: jnp.transpose(a, (0, 2, 1))
    def unpool(x):
        B, N, C = x.shape
        V = N // L_ORI
        return jnp.repeat(x.reshape(B, L_ORI, V, C), 2, axis=2).reshape(B, 2*N, C)
    eis = [edge_index1, edge_index2, edge_index3, edge_index4, edge_index5]
    Ws = [(W1_1, W1_2, S1), (W2_1, W2_2, S2), (W3_1, W3_2, S3), (W4_1, W4_2, S4), (W5_1, W5_2, S5)]
    encs = [x_enc1, x_enc2, x_enc3, x_enc4, x_enc5]
    x = jnp.concatenate([unpool(t(x_enc0)), t(x_enc1)], axis=2)
    for b in range(1, 6):
        n = N_LVL[b]
        W1, W2, S = Ws[b - 1]
        xd = _resblock(x[0], eis[b - 1], n, W1, W2, S)[None]
        if b < 5:
            x = jnp.concatenate([unpool(xd), t(encs[b])], axis=2)
    B, N, C = xd.shape
    V = N // L_ORI
    xp = xd.reshape(B, L_ORI, V, C).max(axis=1)
    out = jnp.transpose(jnp.einsum('bnc,co->bno', xp, Wc), (0, 2, 1))
    return jax.nn.log_softmax(out, axis=1)


# speedup vs baseline: 5.1599x; 2.3950x over previous
import functools

import jax
import jax.numpy as jnp
from jax import lax
from jax.experimental import pallas as pl
from jax.experimental.pallas import tpu as pltpu
from jax.experimental.pallas import tpu_sc as plsc

L_ORI = 6
N_LVL = [1536, 3072, 6144, 12288, 24576, 49152]
NC, NS, LN = 2, 16, 16
NW = NC * NS
M_SHIFT = {3072: 5, 6144: 6, 12288: 7, 24576: 8, 49152: 9}

_SC_PARAMS = dict(
    compiler_params=pltpu.CompilerParams(needs_layout_passes=False),
)


def _mesh():
    return plsc.VectorSubcoreMesh(core_axis_name="c", subcore_axis_name="s")


def _bucket_ids(d, m):
    return lax.shift_right_logical((lax.shift_right_logical(d, m) * 21846), 16)


def make_bucket_kernel(N):
    E = 16 * N
    EW = E // NW
    EWP = EW + 256
    CHK = 1536
    NCHK = EW // CHK
    m = M_SHIFT[N]

    @functools.partial(
        pl.kernel, mesh=_mesh(),
        out_type=(
            jax.ShapeDtypeStruct((NW * EWP + 768,), jnp.int32),
            jax.ShapeDtypeStruct((NW * EWP + 768,), jnp.int32),
            jax.ShapeDtypeStruct((NW, 2, 544), jnp.int32),
        ),
        scratch_types=[
            pltpu.VMEM((CHK,), jnp.int32),
            pltpu.VMEM((CHK,), jnp.int32),
            pltpu.VMEM((512,), jnp.int32),
            pltpu.VMEM((544,), jnp.int32),
            pltpu.VMEM((2, 544), jnp.int32),
            pltpu.VMEM((EWP,), jnp.int32),
            pltpu.VMEM((EWP,), jnp.int32),
        ],
        **_SC_PARAMS,
    )
    def k(src_hbm, dst_hbm, stS_hbm, stD_hbm, sb_hbm,
          sbuf, dbuf, hist, fill, sb2, stS, stD):
        c = lax.axis_index("c")
        s = lax.axis_index("s")
        wid = c * NS + s
        base = wid * EW
        lane = lax.iota(jnp.int32, 16)
        zeros16 = jnp.zeros((16,), jnp.int32)
        ones16 = jnp.ones((16,), jnp.int32)

        for j in range(512 // 16):
            hist[pl.ds(j * 16, 16)] = zeros16

        def scan1(j, carry):
            pltpu.sync_copy(dst_hbm.at[pl.ds(base + j * CHK, CHK)], dbuf)

            def inner(g, cc):
                d = dbuf[pl.ds(g * 16, 16)]
                t = _bucket_ids(d, m) * 16 + lane
                plsc.addupdate_scatter(hist, [t], ones16)
                return cc

            lax.fori_loop(0, CHK // 16, inner, 0)
            return carry

        lax.fori_loop(0, NCHK, scan1, 0)

        def csum(j, carry):
            h = hist[pl.ds(j * 16, 16)]
            ex = plsc.cumsum(h) - h
            sb2[0, pl.ds(j * 16, 16)] = ex + carry
            tot = carry + lax.reduce_sum(h, axes=(0,))
            return jnp.bitwise_and(tot + 7, -8)

        total = lax.fori_loop(0, 32, csum, jnp.int32(0))
        sb2[0, pl.ds(512, 16)] = jnp.full((16,), 1, jnp.int32) * total
        for j in range(34):
            fill[pl.ds(j * 16, 16)] = sb2[0, pl.ds(j * 16, 16)]

        def zstage(j, carry):
            stS[pl.ds(j * 16, 16)] = zeros16
            stD[pl.ds(j * 16, 16)] = zeros16
            return carry

        lax.fori_loop(0, EWP // 16, zstage, 0)

        def scan2(j, carry):
            off = base + j * CHK
            pltpu.sync_copy(src_hbm.at[pl.ds(off, CHK)], sbuf)
            pltpu.sync_copy(dst_hbm.at[pl.ds(off, CHK)], dbuf)

            def inner(g, cc):
                d = dbuf[pl.ds(g * 16, 16)]
                sv = sbuf[pl.ds(g * 16, 16)]
                t = _bucket_ids(d, m) * 16 + lane
                slot = plsc.load_gather(fill, [t])
                plsc.store_scatter(stD, [slot], d)
                plsc.store_scatter(stS, [slot], sv)
                plsc.addupdate_scatter(fill, [t], ones16)
                return cc

            lax.fori_loop(0, CHK // 16, inner, 0)
            return carry

        lax.fori_loop(0, NCHK, scan2, 0)

        for j in range(34):
            sb2[1, pl.ds(j * 16, 16)] = fill[pl.ds(j * 16, 16)]

        pltpu.sync_copy(stS, stS_hbm.at[pl.ds(wid * EWP, EWP)])
        pltpu.sync_copy(stD, stD_hbm.at[pl.ds(wid * EWP, EWP)])
        pltpu.sync_copy(sb2, sb_hbm.at[wid])

        @pl.when(wid == NW - 1)
        def _():
            for j in range(768 // 16):
                stS[pl.ds(j * 16, 16)] = zeros16
            pltpu.sync_copy(stS.at[pl.ds(0, 768)], stS_hbm.at[pl.ds(NW * EWP, 768)])
            pltpu.sync_copy(stS.at[pl.ds(0, 768)], stD_hbm.at[pl.ds(NW * EWP, 768)])

    return k


def make_deg_kernel(N):
    E = 16 * N
    EW = E // NW
    EWP = EW + 256
    R = N // NW
    CHKD = 512

    @functools.partial(
        pl.kernel, mesh=_mesh(),
        out_type=jax.ShapeDtypeStruct((N,), jnp.float32),
        scratch_types=[
            pltpu.VMEM((CHKD + 16,), jnp.int32),
            pltpu.VMEM((2, 544), jnp.int32),
            pltpu.VMEM((16 * R,), jnp.float32),
            pltpu.VMEM((R,), jnp.float32),
        ],
        **_SC_PARAMS,
    )
    def k(stD_hbm, sb_hbm, deg_hbm, dbuf, sbl, hist, degl):
        c = lax.axis_index("c")
        s = lax.axis_index("s")
        w = c * NS + s
        wR = w * R
        lane = lax.iota(jnp.int32, 16)
        fone = jnp.ones((16,), jnp.float32)
        zf = jnp.zeros((16,), jnp.float32)

        def zhist(j, carry):
            hist[pl.ds(j * 16, 16)] = zf
            return carry

        lax.fori_loop(0, R, zhist, 0)

        for wp in range(NW):
            pltpu.sync_copy(sb_hbm.at[wp], sbl)
            sA = sbl[0, pl.ds(w * 16, 16)][0]
            endv = sbl[1, pl.ds(w * 16 + 15, 16)][0]
            ln = endv - sA
            gstart = pl.multiple_of(wp * EWP + sA, 8)

            def chunk(j, carry):
                pltpu.sync_copy(stD_hbm.at[pl.ds(gstart + j * CHKD, CHKD)],
                                dbuf.at[pl.ds(0, CHKD)])
                rem = ln - j * CHKD

                def inner(g, cc):
                    d = dbuf[pl.ds(g * 16, 16)]
                    msk = lane < (rem - g * 16)
                    idx = lane * R + (d - wR)
                    plsc.addupdate_scatter(hist, [idx], fone, mask=msk)
                    return cc

                nv = jnp.minimum(rem, CHKD)
                lax.fori_loop(0, (nv + 15) // 16, inner, 0)
                return carry

            lax.fori_loop(0, (ln + CHKD - 1) // CHKD, chunk, 0)

        def red(i, carry):
            acc = hist[pl.ds(i * 16, 16)]
            for l in range(1, 16):
                acc = acc + hist[pl.ds(l * R + i * 16, 16)]
            degl[pl.ds(i * 16, 16)] = acc
            return carry

        lax.fori_loop(0, R // 16, red, 0)
        pltpu.sync_copy(degl, deg_hbm.at[pl.ds(wR, R)])

    return k


def make_pass_kernel(N, C):
    E = 16 * N
    EW = E // NW
    EWP = EW + 256
    R = N // NW
    K = 128
    CP = max(C, 128)

    @functools.partial(
        pl.kernel, mesh=_mesh(),
        out_type=jax.ShapeDtypeStruct((N * C,), jnp.float32),
        scratch_types=[
            pltpu.VMEM((K,), jnp.int32),
            pltpu.VMEM((K + 16,), jnp.int32),
            pltpu.VMEM((K, CP), jnp.float32),
            pltpu.VMEM((R * C,), jnp.float32),
            pltpu.VMEM((2, 544), jnp.int32),
            pltpu.SemaphoreType.DMA,
        ],
        **_SC_PARAMS,
    )
    def k(xs_hbm, stS_hbm, stD_hbm, sb_hbm, zer_hbm, out_hbm,
          sidx, didx, rows, acc, sbl, sem):
        c = lax.axis_index("c")
        s = lax.axis_index("s")
        w = c * NS + s
        wR = w * R

        pltpu.sync_copy(zer_hbm, acc)

        for wp in range(NW):
            pltpu.sync_copy(sb_hbm.at[wp], sbl)
            sA = sbl[0, pl.ds(w * 16, 16)][0]
            endv = sbl[1, pl.ds(w * 16 + 15, 16)][0]
            ln = endv - sA
            gstart = pl.multiple_of(wp * EWP + sA, 8)

            def chunk(j, carry):
                off = gstart + j * K
                pltpu.sync_copy(stS_hbm.at[pl.ds(off, K)], sidx)
                pltpu.sync_copy(stD_hbm.at[pl.ds(off, K)], didx.at[pl.ds(0, K)])
                pltpu.async_copy(xs_hbm.at[sidx], rows, sem).wait()
                nloc = jnp.minimum(ln - j * K, K)

                def ebody(e, cc):
                    dl = (didx[pl.ds(e, 16)][0] - wR) * C
                    for cb in range(C // 16):
                        sl = pl.ds(dl + cb * 16, 16)
                        acc[sl] = acc[sl] + rows[e, pl.ds(cb * 16, 16)]
                    return cc

                lax.fori_loop(0, nloc, ebody, 0)
                return carry

            lax.fori_loop(0, (ln + K - 1) // K, chunk, 0)

        pltpu.sync_copy(acc, out_hbm.at[pl.ds(wR * C, R * C)])

    return k


BN = 512


def _mm_epilogue_kernel(x_ref, w_ref, d_ref, m_ref, s_ref, C, CP):
    m = jnp.dot(x_ref[...], w_ref[...], preferred_element_type=jnp.float32)
    m_ref[...] = m
    v = d_ref[...] * m[:, 2 * C:3 * C]
    if CP > C:
        v = jnp.concatenate(
            [v, jnp.zeros((v.shape[0], CP - C), jnp.float32)], axis=1)
    s_ref[...] = v


def mm_fused(x, wfull, dexp, C):
    N, Cin = x.shape
    W4 = wfull.shape[1]
    CP = max(C, 128)
    grid = (N // BN,)
    return pl.pallas_call(
        functools.partial(_mm_epilogue_kernel, C=C, CP=CP),
        grid=grid,
        in_specs=[
            pl.BlockSpec((BN, Cin), lambda i: (i, 0)),
            pl.BlockSpec((Cin, W4), lambda i: (0, 0)),
            pl.BlockSpec((BN, C), lambda i: (i, 0)),
        ],
        out_specs=[
            pl.BlockSpec((BN, W4), lambda i: (i, 0)),
            pl.BlockSpec((BN, CP), lambda i: (i, 0)),
        ],
        out_shape=[
            jax.ShapeDtypeStruct((N, W4), jnp.float32),
            jax.ShapeDtypeStruct((N, CP), jnp.float32),
        ],
    )(x, wfull, dexp)


def _glue_mid_kernel(m_ref, z_ref, d_ref, o_ref, C, CP):
    d = d_ref[...]
    v = d * (m_ref[:, C:2 * C] - 2.0 * d * z_ref[...])
    if CP > C:
        v = jnp.concatenate(
            [v, jnp.zeros((v.shape[0], CP - C), jnp.float32)], axis=1)
    o_ref[...] = v


def glue_mid(M, zraw, dexp, C):
    N = M.shape[0]
    W = M.shape[1]
    CP = max(C, 128)
    return pl.pallas_call(
        functools.partial(_glue_mid_kernel, C=C, CP=CP),
        grid=(N // BN,),
        in_specs=[
            pl.BlockSpec((BN, W), lambda i: (i, 0)),
            pl.BlockSpec((BN, C), lambda i: (i, 0)),
            pl.BlockSpec((BN, C), lambda i: (i, 0)),
        ],
        out_specs=pl.BlockSpec((BN, CP), lambda i: (i, 0)),
        out_shape=jax.ShapeDtypeStruct((N, CP), jnp.float32),
    )(M, zraw, dexp)


def _glue_h_kernel(m_ref, u_ref, d_ref, o_ref, C):
    o_ref[...] = jnp.maximum(
        m_ref[:, 0:C] - m_ref[:, 2 * C:3 * C] - d_ref[...] * u_ref[...], 0.0)


def glue_h(M, uraw, dexp, C):
    N = M.shape[0]
    W = M.shape[1]
    return pl.pallas_call(
        functools.partial(_glue_h_kernel, C=C),
        grid=(N // BN,),
        in_specs=[
            pl.BlockSpec((BN, W), lambda i: (i, 0)),
            pl.BlockSpec((BN, C), lambda i: (i, 0)),
            pl.BlockSpec((BN, C), lambda i: (i, 0)),
        ],
        out_specs=pl.BlockSpec((BN, C), lambda i: (i, 0)),
        out_shape=jax.ShapeDtypeStruct((N, C), jnp.float32),
    )(M, uraw, dexp)


def _glue_out_kernel(m2_ref, u_ref, d_ref, r_ref, o_ref, C):
    o_ref[...] = jnp.maximum(
        m2_ref[:, 0:C] - m2_ref[:, 2 * C:3 * C] - d_ref[...] * u_ref[...]
        + r_ref[...], 0.0)


def glue_out(M2, u2raw, dexp, res, C):
    N = M2.shape[0]
    W = M2.shape[1]
    return pl.pallas_call(
        functools.partial(_glue_out_kernel, C=C),
        grid=(N // BN,),
        in_specs=[
            pl.BlockSpec((BN, W), lambda i: (i, 0)),
            pl.BlockSpec((BN, C), lambda i: (i, 0)),
            pl.BlockSpec((BN, C), lambda i: (i, 0)),
            pl.BlockSpec((BN, C), lambda i: (i, 0)),
        ],
        out_specs=pl.BlockSpec((BN, C), lambda i: (i, 0)),
        out_shape=jax.ShapeDtypeStruct((N, C), jnp.float32),
    )(M2, u2raw, dexp, res)


def _head_kernel(x_ref, w_ref, o_ref):
    xp = x_ref[0]
    for i in range(1, L_ORI):
        xp = jnp.maximum(xp, x_ref[i])
    logits = jnp.dot(xp, w_ref[...], preferred_element_type=jnp.float32)
    mx = jnp.max(logits, axis=1, keepdims=True)
    ex = jnp.exp(logits - mx)
    lse = jnp.log(jnp.sum(ex, axis=1, keepdims=True)) + mx
    o_ref[...] = logits - lse


def head(x6, wc):
    V = x6.shape[1]
    CO = wc.shape[1]
    C = x6.shape[2]
    return pl.pallas_call(
        _head_kernel,
        grid=(V // BN,),
        in_specs=[
            pl.BlockSpec((L_ORI, BN, C), lambda i: (0, i, 0)),
            pl.BlockSpec((C, CO), lambda i: (0, 0)),
        ],
        out_specs=pl.BlockSpec((BN, CO), lambda i: (i, 0)),
        out_shape=jax.ShapeDtypeStruct((V, CO), jnp.float32),
    )(x6, wc)


def _resblock(x, ei, N, W1, W2, S):
    Cin = x.shape[1]
    C = W1.shape[2]
    src, dst = ei[0], ei[1]

    bk = make_bucket_kernel(N)
    stS, stD, sb = bk(src, dst)
    deg = make_deg_kernel(N)(stD, sb)

    dinv = jnp.where(deg > 0, lax.rsqrt(jnp.maximum(deg, 1e-12)), 0.0)
    dexp = jnp.broadcast_to(dinv[:, None], (N, C))
    zer = jnp.zeros(((N // NW) * C,), jnp.float32)

    pk = make_pass_kernel(N, C)

    wfull1 = jnp.concatenate([W1[0], W1[1], W1[2], S], axis=1)
    M1, sy2 = mm_fused(x, wfull1, dexp, C)
    zraw = pk(sy2, stS, stD, sb, zer).reshape(N, C)
    su = glue_mid(M1, zraw, dexp, C)
    uraw = pk(su, stS, stD, sb, zer).reshape(N, C)
    h = glue_h(M1, uraw, dexp, C)

    wfull2 = jnp.concatenate([W2[0], W2[1], W2[2]], axis=1)
    wfull2 = jnp.concatenate(
        [wfull2, jnp.zeros((C, C), jnp.float32)], axis=1)
    M2, st2 = mm_fused(h, wfull2, dexp, C)
    z2 = pk(st2, stS, stD, sb, zer).reshape(N, C)
    su2 = glue_mid(M2, z2, dexp, C)
    u2 = pk(su2, stS, stD, sb, zer).reshape(N, C)
    return glue_out(M2, u2, dexp, M1[:, 3 * C:4 * C], C)


def kernel(x_enc0, x_enc1, x_enc2, x_enc3, x_enc4, x_enc5,
           edge_index1, edge_index2, edge_index3, edge_index4, edge_index5,
           W1_1, W1_2, S1, W2_1, W2_2, S2, W3_1, W3_2, S3,
           W4_1, W4_2, S4, W5_1, W5_2, S5, Wc):
    t2 = lambda a: jnp.transpose(a[0], (1, 0))

    def unpool(x):
        n, C = x.shape
        V = n // L_ORI
        return jnp.repeat(x.reshape(L_ORI, V, C), 2, axis=1).reshape(2 * n, C)

    eis = [edge_index1, edge_index2, edge_index3, edge_index4, edge_index5]
    Ws = [(W1_1, W1_2, S1), (W2_1, W2_2, S2), (W3_1, W3_2, S3),
          (W4_1, W4_2, S4), (W5_1, W5_2, S5)]
    encs = [x_enc1, x_enc2, x_enc3, x_enc4, x_enc5]

    x = jnp.concatenate([unpool(t2(x_enc0)), t2(x_enc1)], axis=1)
    for b in range(1, 6):
        N = N_LVL[b]
        W1, W2, S = Ws[b - 1]
        xd = _resblock(x, eis[b - 1], N, W1, W2, S)
        if b < 5:
            x = jnp.concatenate([unpool(xd), t2(encs[b])], axis=1)

    N, C = xd.shape
    V = N // L_ORI
    x6 = xd.reshape(L_ORI, V, C)
    logp = head(x6, Wc)
    return jnp.transpose(logp, (1, 0))[None]
